# baseline (speedup 1.0000x reference)
import functools

import jax
import jax.numpy as jnp
from jax import lax
from jax.experimental import pallas as pl
from jax.experimental.pallas import tpu as pltpu
from jax.experimental.pallas import tpu_sc as plsc

_NC = 2
_NS = 16
_NW = _NC * _NS
_BB = 128
_L = 16


def _gather_kernel(idx_hbm, table_hbm, out_hbm, idx_v, bufs, bufts, sem_g, sem_w):
    hist = idx_v.shape[0]
    embed = bufs.shape[2]
    wid = lax.axis_index("s") * _NC + lax.axis_index("c")
    b0 = wid * _BB
    pltpu.sync_copy(idx_hbm.at[:, pl.ds(b0, _BB)], idx_v)

    rows_c = [lax.iota(jnp.int32, 16) + (_L * c) for c in range(_BB // _L)]
    zero16 = jnp.zeros((16,), jnp.int32)

    def fire_g(slot, u):
        pltpu.async_copy(table_hbm.at[idx_v.at[u]], bufs.at[slot], sem_g.at[slot])

    def wait_g(slot, u):
        pltpu.make_async_copy(
            table_hbm.at[idx_v.at[u]], bufs.at[slot], sem_g.at[slot]
        ).wait()

    def transpose(slot):
        src = bufs.at[slot]
        dst = bufts.at[slot]

        @plsc.parallel_loop(0, embed, unroll=4)
        def _t(e):
            cols = zero16 + e
            vs = [
                plsc.load_gather(src, [rows_c[c], cols])
                for c in range(_BB // _L)
            ]
            for c in range(_BB // _L):
                dst[e, pl.ds(_L * c, _L)] = vs[c]

    def fire_w(slot, u):
        for eb in range(embed // 8):
            pltpu.async_copy(
                bufts.at[slot].at[pl.ds(eb * 8, 8)],
                out_hbm.at[u, eb, wid],
                sem_w.at[slot],
            )

    def wait_w(slot, u):
        for eb in range(embed // 8):
            pltpu.make_async_copy(
                bufts.at[slot].at[pl.ds(eb * 8, 8)],
                out_hbm.at[u, eb, wid],
                sem_w.at[slot],
            ).wait()

    fire_g(0, 0)
    fire_g(1, 1)

    @pl.loop(0, hist, step=2)
    def _steady(u):
        for slot in range(2):
            uu = u + slot
            wait_g(slot, uu)

            @pl.when(uu >= 2)
            def _():
                wait_w(slot, uu - 2)

            transpose(slot)

            @pl.when(uu + 2 < hist)
            def _():
                fire_g(slot, uu + 2)

            fire_w(slot, uu)

    wait_w(0, hist - 2)
    wait_w(1, hist - 1)


def kernel(input, table):
    batch, hist = input.shape
    _, embed_dim = table.shape
    assert batch == _NW * _BB and hist % 2 == 0 and embed_dim % 8 == 0

    run = functools.partial(
        pl.kernel,
        out_type=jax.ShapeDtypeStruct(
            (hist, embed_dim // 8, _NW, 8, _BB), table.dtype
        ),
        mesh=plsc.VectorSubcoreMesh(core_axis_name="c", subcore_axis_name="s"),
        scratch_types=[
            pltpu.VMEM((hist, _BB), jnp.int32),
            pltpu.VMEM((2, _BB, embed_dim), jnp.float32),
            pltpu.VMEM((2, embed_dim, _BB), jnp.float32),
            pltpu.SemaphoreType.DMA((2,)),
            pltpu.SemaphoreType.DMA((2,)),
        ],
        compiler_params=pltpu.CompilerParams(
            use_tc_tiling_on_sc=False, needs_layout_passes=False
        ),
    )(_gather_kernel)

    out5 = run(input.T, table)
    return out5.transpose(2, 4, 0, 1, 3).reshape(batch, hist, embed_dim)


# speedup vs baseline: 1.1495x; 1.1495x over previous
import functools

import jax
import jax.numpy as jnp
from jax import lax
from jax.experimental import pallas as pl
from jax.experimental.pallas import tpu as pltpu
from jax.experimental.pallas import tpu_sc as plsc

_NC = 2
_NS = 16
_NW = _NC * _NS
_BB = 128
_L = 16


def _gather_kernel(idx_hbm, table_hbm, out_hbm, idx_v, bufs, bufts, sem_g, sem_w):
    hist = idx_v.shape[0]
    embed = bufs.shape[2]
    wid = lax.axis_index("s") * _NC + lax.axis_index("c")
    b0 = wid * _BB
    pltpu.sync_copy(idx_hbm.at[:, pl.ds(b0, _BB)], idx_v)

    erows_c = [lax.iota(jnp.int32, 16) + (_L * c) for c in range(4)]
    zero16 = jnp.zeros((16,), jnp.int32)

    def fire_g(slot, u):
        pltpu.async_copy(table_hbm.at[idx_v.at[u]], bufs.at[slot], sem_g.at[slot])

    def wait_g(slot, u):
        pltpu.make_async_copy(
            table_hbm.at[idx_v.at[u]], bufs.at[slot], sem_g.at[slot]
        ).wait()

    def transpose(slot):
        src = bufs.at[slot]
        dst = bufts.at[slot]

        @plsc.parallel_loop(0, _BB, unroll=4)
        def _t(b):
            bcol = zero16 + b
            for ec in range(embed // _L):
                v = src[b, pl.ds(_L * ec, _L)]
                plsc.store_scatter(dst, [erows_c[ec], bcol], v)

    def fire_w(slot, u):
        for eb in range(embed // 8):
            pltpu.async_copy(
                bufts.at[slot].at[pl.ds(eb * 8, 8)],
                out_hbm.at[u, eb, wid],
                sem_w.at[slot],
            )

    def wait_w(slot, u):
        for eb in range(embed // 8):
            pltpu.make_async_copy(
                bufts.at[slot].at[pl.ds(eb * 8, 8)],
                out_hbm.at[u, eb, wid],
                sem_w.at[slot],
            ).wait()

    fire_g(0, 0)
    fire_g(1, 1)

    @pl.loop(0, hist, step=2)
    def _steady(u):
        for slot in range(2):
            uu = u + slot
            wait_g(slot, uu)

            @pl.when(uu >= 2)
            def _():
                wait_w(slot, uu - 2)

            transpose(slot)

            @pl.when(uu + 2 < hist)
            def _():
                fire_g(slot, uu + 2)

            fire_w(slot, uu)

    wait_w(0, hist - 2)
    wait_w(1, hist - 1)


def kernel(input, table):
    batch, hist = input.shape
    _, embed_dim = table.shape
    assert batch == _NW * _BB and hist % 2 == 0 and embed_dim % 8 == 0

    run = functools.partial(
        pl.kernel,
        out_type=jax.ShapeDtypeStruct(
            (hist, embed_dim // 8, _NW, 8, _BB), table.dtype
        ),
        mesh=plsc.VectorSubcoreMesh(core_axis_name="c", subcore_axis_name="s"),
        scratch_types=[
            pltpu.VMEM((hist, _BB), jnp.int32),
            pltpu.VMEM((2, _BB, embed_dim), jnp.float32),
            pltpu.VMEM((2, embed_dim, _BB), jnp.float32),
            pltpu.SemaphoreType.DMA((2,)),
            pltpu.SemaphoreType.DMA((2,)),
        ],
        compiler_params=pltpu.CompilerParams(
            use_tc_tiling_on_sc=False, needs_layout_passes=False
        ),
    )(_gather_kernel)

    out5 = run(input.T, table)
    return out5.transpose(2, 4, 0, 1, 3).reshape(batch, hist, embed_dim)


# speedup vs baseline: 1.2217x; 1.0628x over previous
import functools

import jax
import jax.numpy as jnp
from jax import lax
from jax.experimental import pallas as pl
from jax.experimental.pallas import tpu as pltpu
from jax.experimental.pallas import tpu_sc as plsc

_NC = 2
_NS = 16
_NW = _NC * _NS
_NBUF = 2


def _gather_kernel(idx_hbm, table_hbm, out_hbm, idx_v, bufs, sem_g, sem_w):
    b_per_w, hist = idx_v.shape
    nr = b_per_w // _NBUF
    wid = lax.axis_index("s") * _NC + lax.axis_index("c")
    b0 = wid * b_per_w
    pltpu.sync_copy(idx_hbm.at[pl.ds(b0, b_per_w)], idx_v)

    def idx_slice(t):
        return idx_v.at[t]

    def out_slice(t):
        return out_hbm.at[b0 + t]

    def fire_g(slot, t):
        pltpu.async_copy(table_hbm.at[idx_slice(t)], bufs.at[slot], sem_g.at[slot])

    def wait_g(slot, t):
        pltpu.make_async_copy(
            table_hbm.at[idx_slice(t)], bufs.at[slot], sem_g.at[slot]
        ).wait()

    def fire_w(slot, t):
        pltpu.async_copy(bufs.at[slot], out_slice(t), sem_w.at[slot])

    def wait_w(slot, t):
        pltpu.make_async_copy(bufs.at[slot], out_slice(t), sem_w.at[slot]).wait()

    for b in range(_NBUF):
        fire_g(b, b)
    for b in range(_NBUF):
        wait_g(b, b)
        fire_w(b, b)
    for b in range(_NBUF):
        fire_g(_NBUF + b, _NBUF + b)

    @pl.loop(1, nr - 1, step=2)
    def _steady(r):
        for b in range(_NBUF):
            wait_g(_NBUF + b, r * _NBUF + b)
            fire_w(_NBUF + b, r * _NBUF + b)
        for b in range(_NBUF):
            wait_w(b, (r - 1) * _NBUF + b)
            fire_g(b, (r + 1) * _NBUF + b)
        for b in range(_NBUF):
            wait_g(b, (r + 1) * _NBUF + b)
            fire_w(b, (r + 1) * _NBUF + b)
        for b in range(_NBUF):
            wait_w(_NBUF + b, r * _NBUF + b)
            fire_g(_NBUF + b, (r + 2) * _NBUF + b)

    for b in range(_NBUF):
        wait_g(_NBUF + b, (nr - 1) * _NBUF + b)
        fire_w(_NBUF + b, (nr - 1) * _NBUF + b)
    for b in range(_NBUF):
        wait_w(b, (nr - 2) * _NBUF + b)
    for b in range(_NBUF):
        wait_w(_NBUF + b, (nr - 1) * _NBUF + b)


def kernel(input, table):
    batch, hist = input.shape
    _, embed_dim = table.shape
    assert batch % _NW == 0
    b_per_w = batch // _NW
    assert b_per_w % (2 * _NBUF) == 0

    run = functools.partial(
        pl.kernel,
        out_type=jax.ShapeDtypeStruct((batch, hist, embed_dim), table.dtype),
        mesh=plsc.VectorSubcoreMesh(core_axis_name="c", subcore_axis_name="s"),
        scratch_types=[
            pltpu.VMEM((b_per_w, hist), jnp.int32),
            pltpu.VMEM((2 * _NBUF, hist, embed_dim), jnp.float32),
            pltpu.SemaphoreType.DMA((2 * _NBUF,)),
            pltpu.SemaphoreType.DMA((2 * _NBUF,)),
        ],
        compiler_params=pltpu.CompilerParams(use_tc_tiling_on_sc=False),
    )(_gather_kernel)

    return run(input, table)
